# baseline (speedup 1.0000x reference)
import functools

import jax
import jax.numpy as jnp
from jax import lax
from jax.experimental import pallas as pl
from jax.experimental.pallas import tpu as pltpu
from jax.experimental.pallas import tpu_sc as plsc

N_NODES = 100000
N_EDGES = 1600000
B = 10000
NBR = 20
NODE_DIM = 128
EDGE_DIM = 16
TIME_DIM = 16
QDIM = NODE_DIM + TIME_DIM
N_HEADS = 2
DH = QDIM // N_HEADS
EPR = NODE_DIM // EDGE_DIM

NW = 32
NODE_TOT = 215040
NODE_PW = NODE_TOT // NW
NODE_CH = 448
NODE_NCH = NODE_PW // NODE_CH
EW_TOT = 204800
EW_PW = EW_TOT // NW
EW_CH = 400
EW_NCH = EW_PW // EW_CH

RB = 80
NRB = RB * NBR
NBLK = B // RB


def _combine_body(m_ref, f_ref, o_ref):
    o_ref[...] = m_ref[...] + f_ref[...]


def _sc_node_body(comb_hbm, nidx_hbm, nrows_out, nidx_v, nbuf0, nbuf1,
                  nsem0, nsem1):
    wid = lax.axis_index("s") * 2 + lax.axis_index("c")
    nbase = wid * NODE_PW
    pltpu.sync_copy(nidx_hbm.at[pl.ds(nbase, NODE_PW)], nidx_v)
    bufs, sems = (nbuf0, nbuf1), (nsem0, nsem1)
    for p in range(2):
        pltpu.async_copy(
            comb_hbm.at[nidx_v.at[pl.ds(p * NODE_CH, NODE_CH)]],
            bufs[p], sems[p])

    @pl.loop(0, NODE_NCH + 1, step=2)
    def _(g):
        for p in range(2):
            c = g + p

            @pl.when(c < NODE_NCH)
            def _node():
                buf, sem = bufs[p], sems[p]
                pltpu.make_async_copy(
                    comb_hbm.at[pl.ds(0, NODE_CH)], buf, sem).wait()
                pltpu.sync_copy(
                    buf, nrows_out.at[pl.ds(nbase + c * NODE_CH, NODE_CH)])

                @pl.when(c + 2 < NODE_NCH)
                def _issue():
                    off = (c + 2) * NODE_CH
                    pltpu.async_copy(
                        comb_hbm.at[nidx_v.at[pl.ds(off, NODE_CH)]], buf, sem)


def _sc_edge_body(efw_hbm, eidx_hbm, erows_out, eidx_v, ebuf0, ebuf1,
                  esem0, esem1):
    wid = lax.axis_index("s") * 2 + lax.axis_index("c")
    ebase = wid * EW_PW
    pltpu.sync_copy(eidx_hbm.at[pl.ds(ebase, EW_PW)], eidx_v)
    bufs, sems = (ebuf0, ebuf1), (esem0, esem1)
    for p in range(2):
        pltpu.async_copy(
            efw_hbm.at[eidx_v.at[pl.ds(p * EW_CH, EW_CH)]], bufs[p], sems[p])

    @pl.loop(0, EW_NCH, step=2)
    def _(g):
        for p in range(2):
            c = g + p
            buf, sem = bufs[p], sems[p]
            pltpu.make_async_copy(
                efw_hbm.at[pl.ds(0, EW_CH)], buf, sem).wait()
            pltpu.sync_copy(
                buf, erows_out.at[pl.ds(ebase + c * EW_CH, EW_CH)])

            @pl.when(c + 2 < EW_NCH)
            def _issue():
                off = (c + 2) * EW_CH
                pltpu.async_copy(
                    efw_hbm.at[eidx_v.at[pl.ds(off, EW_CH)]], buf, sem)


def _attn_body(nbr_ref, src_ref, efw_ref, et_ref, ts_ref, mk_ref, g_ref,
               tw_ref, tb_ref,
               wq1_ref, wq2_ref, bq_ref,
               wk_ref, bk_ref, wv_ref, bv_ref,
               wo_ref, bo_ref,
               fc1a_ref, fc1b_ref, fc1b_b_ref, fc2_ref, fc2b_ref,
               out_ref):
    f32 = jnp.float32
    i32 = jnp.int32
    rows = lax.broadcasted_iota(i32, (NRB, RB), 0) // NBR
    cols = lax.broadcasted_iota(i32, (NRB, RB), 1)
    bd = (rows == cols).astype(f32)
    rows_t = lax.broadcasted_iota(i32, (RB, NRB), 1) // NBR
    cols_t = lax.broadcasted_iota(i32, (RB, NRB), 0)
    bdt = (rows_t == cols_t).astype(f32)
    dsel = lax.broadcasted_iota(i32, (QDIM, N_HEADS), 0) // DH
    hsel = lax.broadcasted_iota(i32, (QDIM, N_HEADS), 1)
    seg = (dsel == hsel).astype(f32)
    dsel2 = lax.broadcasted_iota(i32, (N_HEADS, QDIM), 1) // DH
    hsel2 = lax.broadcasted_iota(i32, (N_HEADS, QDIM), 0)
    seg_t = (dsel2 == hsel2).astype(f32)

    dotH = functools.partial(jnp.dot, preferred_element_type=f32,
                             precision=lax.Precision.HIGHEST)
    dotD = functools.partial(jnp.dot, preferred_element_type=f32)

    lanesel = (lax.broadcasted_iota(i32, (NRB, NBR), 1)
               == lax.broadcasted_iota(i32, (NRB, NBR), 0) % NBR).astype(f32)
    ones_col = jnp.ones((NBR, 1), f32)

    def expand_col(x, dot):
        return dot(dot(bd, x) * lanesel, ones_col)

    nbr = nbr_ref[...]
    src = src_ref[...]
    efw = efw_ref[...]
    et = expand_col(et_ref[...], dotH)
    mask_add = expand_col(mk_ref[...], dotH)
    gcol = expand_col(g_ref[...], dotD)
    ts = ts_ref[...]

    lane = lax.broadcasted_iota(i32, (NRB, NODE_DIM), 1)
    ef_sel = jnp.where((lane // EDGE_DIM).astype(f32) == gcol, efw, 0.0)
    psel = (lax.broadcasted_iota(i32, (NODE_DIM, EDGE_DIM), 0) % EDGE_DIM
            == lax.broadcasted_iota(i32, (NODE_DIM, EDGE_DIM), 1))
    ef = dotD(ef_sel, psel.astype(f32))

    deltas = dotH(bd, ts) - et
    et_enc = jnp.cos(deltas * tw_ref[...] + tb_ref[...])
    st_row = jnp.cos(tb_ref[...])

    q = dotH(src, wq1_ref[...]) + dotH(st_row, wq2_ref[...]) + bq_ref[...]
    keyk = jnp.concatenate([nbr, et_enc, ef], axis=1)
    k = dotH(keyk, wk_ref[...]) + bk_ref[...]
    v = dotH(keyk, wv_ref[...]) + bv_ref[...]

    q_rep = dotD(bd, q)
    scores = dotD(q_rep * k, seg) * (1.0 / (DH ** 0.5))
    scores = scores + mask_add
    e = jnp.exp(scores)
    den = dotD(bdt, e)
    fully_masked = den[:, 0:1] == 0.0
    den = jnp.where(den == 0.0, 1.0, den)
    attn = e * dotD(bd, 1.0 / den)
    av = dotD(attn, seg_t) * v
    outh = dotD(bdt, av)
    out = dotH(outh, wo_ref[...]) + bo_ref[...]
    out = jnp.where(fully_masked, 0.0, out)

    h1 = jnp.maximum(
        dotH(out, fc1a_ref[...]) + dotH(src, fc1b_ref[...]) + fc1b_b_ref[...],
        0.0)
    out_ref[...] = dotH(h1, fc2_ref[...]) + fc2b_ref[...]


def kernel(memory, node_features, edge_features, timestamps, edge_times,
           time_w, time_b, Wq, bq, Wk, bk, Wv, bv, Wo, bo,
           fc1_w, fc1_b, fc2_w, fc2_b, src_nodes, neighbors, edge_idxs):
    f32 = jnp.float32

    combined = pl.pallas_call(
        _combine_body,
        out_shape=jax.ShapeDtypeStruct((N_NODES, NODE_DIM), f32),
        grid=(50,),
        in_specs=[pl.BlockSpec((2000, NODE_DIM), lambda i: (i, 0)),
                  pl.BlockSpec((2000, NODE_DIM), lambda i: (i, 0))],
        out_specs=pl.BlockSpec((2000, NODE_DIM), lambda i: (i, 0)),
    )(memory, node_features)

    flat_nbr = neighbors.reshape(-1).astype(jnp.int32)
    flat_eix = edge_idxs.reshape(-1).astype(jnp.int32)
    node_idx = jnp.concatenate([
        flat_nbr, src_nodes.astype(jnp.int32),
        jnp.zeros((NODE_TOT - B * NBR - B,), jnp.int32)])
    ew_idx = jnp.concatenate([
        flat_eix // EPR, jnp.zeros((EW_TOT - B * NBR,), jnp.int32)])
    efw = edge_features.reshape(N_EDGES // EPR, NODE_DIM)

    mesh = plsc.VectorSubcoreMesh(core_axis_name="c", subcore_axis_name="s")
    node_rows = pl.kernel(
        _sc_node_body,
        out_type=jax.ShapeDtypeStruct((NODE_TOT, NODE_DIM), f32),
        mesh=mesh,
        scratch_types=[
            pltpu.VMEM((NODE_PW,), jnp.int32),
            pltpu.VMEM((NODE_CH, NODE_DIM), f32),
            pltpu.VMEM((NODE_CH, NODE_DIM), f32),
            pltpu.SemaphoreType.DMA,
            pltpu.SemaphoreType.DMA,
        ],
    )(combined, node_idx)

    ew_rows = pl.kernel(
        _sc_edge_body,
        out_type=jax.ShapeDtypeStruct((EW_TOT, NODE_DIM), f32),
        mesh=mesh,
        scratch_types=[
            pltpu.VMEM((EW_PW,), jnp.int32),
            pltpu.VMEM((EW_CH, NODE_DIM), f32),
            pltpu.VMEM((EW_CH, NODE_DIM), f32),
            pltpu.SemaphoreType.DMA,
            pltpu.SemaphoreType.DMA,
        ],
    )(efw, ew_idx)

    ts_col = timestamps.reshape(B, 1).astype(f32)
    et2d = edge_times.astype(f32)
    mask2d = jnp.where(neighbors == 0, -1e9, 0.0).astype(f32)
    g2d = (flat_eix.reshape(B, NBR) % EPR).astype(f32)

    tw = time_w.reshape(1, TIME_DIM)
    tb = time_b.reshape(1, TIME_DIM)
    wq1 = Wq[:, :NODE_DIM].T
    wq2 = Wq[:, NODE_DIM:].T
    wk_t = Wk.T
    wv_t = Wv.T
    wo_t = Wo.T
    fc1a = fc1_w[:, :QDIM].T
    fc1b = fc1_w[:, QDIM:].T
    fc2t = fc2_w.T

    def full(a):
        a2 = a.reshape((1, -1)) if a.ndim == 1 else a
        return a2, pl.BlockSpec(a2.shape, lambda i: tuple(0 for _ in a2.shape))

    const_args = [tw, tb, wq1, wq2, bq, wk_t, bk, wv_t, bv,
                  wo_t, bo, fc1a, fc1b, fc1_b, fc2t, fc2_b]
    const_vals, const_specs = zip(*[full(a) for a in const_args])

    out = pl.pallas_call(
        _attn_body,
        out_shape=jax.ShapeDtypeStruct((B, NODE_DIM), f32),
        grid=(NBLK,),
        in_specs=[
            pl.BlockSpec((NRB, NODE_DIM), lambda i: (i, 0)),
            pl.BlockSpec((RB, NODE_DIM), lambda i: (B * NBR // RB + i, 0)),
            pl.BlockSpec((NRB, NODE_DIM), lambda i: (i, 0)),
            pl.BlockSpec((RB, NBR), lambda i: (i, 0)),
            pl.BlockSpec((RB, 1), lambda i: (i, 0)),
            pl.BlockSpec((RB, NBR), lambda i: (i, 0)),
            pl.BlockSpec((RB, NBR), lambda i: (i, 0)),
        ] + list(const_specs),
        out_specs=pl.BlockSpec((RB, NODE_DIM), lambda i: (i, 0)),
    )(node_rows, node_rows, ew_rows, et2d, ts_col, mask2d, g2d,
      *const_vals)
    return out


# speedup vs baseline: 1.1599x; 1.1599x over previous
import functools

import jax
import jax.numpy as jnp
from jax import lax
from jax.experimental import pallas as pl
from jax.experimental.pallas import tpu as pltpu
from jax.experimental.pallas import tpu_sc as plsc

N_NODES = 100000
N_EDGES = 1600000
B = 10000
NBR = 20
NODE_DIM = 128
EDGE_DIM = 16
TIME_DIM = 16
QDIM = NODE_DIM + TIME_DIM
N_HEADS = 2
DH = QDIM // N_HEADS
EPR = NODE_DIM // EDGE_DIM

NW = 32
SC_CH = 320
NODE_TOT = 215040
NODE_N0 = 10560
NODE_N1 = 2880
EW_TOT = 204800
EW_N0 = 9920
EW_N1 = 2880

RB = 80
NRB = RB * NBR
NBLK = B // RB


def _combine_body(m_ref, f_ref, o_ref):
    o_ref[...] = m_ref[...] + f_ref[...]


def _make_sc_body(n0, n1):
    nch_max = n0 // SC_CH

    def body(table_hbm, idx_hbm, out, idx_v, buf0, buf1, sem0, sem1):
        c = lax.axis_index("c")
        s = lax.axis_index("s")
        base = jnp.where(c == 0, s * n0, 16 * n0 + s * n1)
        nch = jnp.where(c == 0, n0 // SC_CH, n1 // SC_CH)
        pltpu.sync_copy(idx_hbm.at[pl.ds(base, n0)], idx_v)
        bufs, sems = (buf0, buf1), (sem0, sem1)
        for p in range(2):
            pltpu.async_copy(
                table_hbm.at[idx_v.at[pl.ds(p * SC_CH, SC_CH)]],
                bufs[p], sems[p])

        @pl.loop(0, nch_max + 1, step=2)
        def _(g):
            for p in range(2):
                cch = g + p

                @pl.when(cch < nch)
                def _do():
                    buf, sem = bufs[p], sems[p]
                    pltpu.make_async_copy(
                        table_hbm.at[pl.ds(0, SC_CH)], buf, sem).wait()
                    pltpu.sync_copy(
                        buf, out.at[pl.ds(base + cch * SC_CH, SC_CH)])

                    @pl.when(cch + 2 < nch)
                    def _issue():
                        off = (cch + 2) * SC_CH
                        pltpu.async_copy(
                            table_hbm.at[idx_v.at[pl.ds(off, SC_CH)]],
                            buf, sem)

    return body


_sc_node_body = _make_sc_body(NODE_N0, NODE_N1)
_sc_edge_body = _make_sc_body(EW_N0, EW_N1)


def _attn_body(nbr_ref, src_ref, efw_ref, et_ref, ts_ref, mk_ref, g_ref,
               tw_ref, tb_ref,
               wq1_ref, wq2_ref, bq_ref,
               wk_ref, bk_ref, wv_ref, bv_ref,
               wo_ref, bo_ref,
               fc1a_ref, fc1b_ref, fc1b_b_ref, fc2_ref, fc2b_ref,
               out_ref):
    f32 = jnp.float32
    i32 = jnp.int32
    rows = lax.broadcasted_iota(i32, (NRB, RB), 0) // NBR
    cols = lax.broadcasted_iota(i32, (NRB, RB), 1)
    bd = (rows == cols).astype(f32)
    rows_t = lax.broadcasted_iota(i32, (RB, NRB), 1) // NBR
    cols_t = lax.broadcasted_iota(i32, (RB, NRB), 0)
    bdt = (rows_t == cols_t).astype(f32)
    dsel = lax.broadcasted_iota(i32, (QDIM, N_HEADS), 0) // DH
    hsel = lax.broadcasted_iota(i32, (QDIM, N_HEADS), 1)
    seg = (dsel == hsel).astype(f32)
    dsel2 = lax.broadcasted_iota(i32, (N_HEADS, QDIM), 1) // DH
    hsel2 = lax.broadcasted_iota(i32, (N_HEADS, QDIM), 0)
    seg_t = (dsel2 == hsel2).astype(f32)

    dotH = functools.partial(jnp.dot, preferred_element_type=f32,
                             precision=lax.Precision.HIGHEST)
    dotD = functools.partial(jnp.dot, preferred_element_type=f32)

    lanesel = (lax.broadcasted_iota(i32, (NRB, NBR), 1)
               == lax.broadcasted_iota(i32, (NRB, NBR), 0) % NBR)

    def expand_col(x):
        e = jnp.where(lanesel, dotD(bd, x), 0.0)
        return jnp.sum(e, axis=1, keepdims=True)

    def split_hi(x):
        return x.astype(jnp.bfloat16).astype(f32)

    nbr = nbr_ref[...]
    src = src_ref[...]
    efw = efw_ref[...]
    et2 = et_ref[...]
    et_hi = split_hi(et2)
    et = expand_col(et_hi) + expand_col(et2 - et_hi)
    mask_add = expand_col(mk_ref[...])
    gcol = expand_col(g_ref[...])
    ts2 = ts_ref[...]
    ts_hi = split_hi(ts2)
    ts_rep = dotD(bd, ts_hi) + dotD(bd, ts2 - ts_hi)

    lane = lax.broadcasted_iota(i32, (NRB, NODE_DIM), 1)
    ef_sel = jnp.where((lane // EDGE_DIM).astype(f32) == gcol, efw, 0.0)
    psel = (lax.broadcasted_iota(i32, (NODE_DIM, EDGE_DIM), 0) % EDGE_DIM
            == lax.broadcasted_iota(i32, (NODE_DIM, EDGE_DIM), 1))
    ef = dotD(ef_sel, psel.astype(f32))

    deltas = ts_rep - et
    et_enc = jnp.cos(deltas * tw_ref[...] + tb_ref[...])
    st_row = jnp.cos(tb_ref[...])

    q = dotH(src, wq1_ref[...]) + dotH(st_row, wq2_ref[...]) + bq_ref[...]
    keyk = jnp.concatenate([nbr, et_enc, ef], axis=1)
    k = dotH(keyk, wk_ref[...]) + bk_ref[...]
    v = dotH(keyk, wv_ref[...]) + bv_ref[...]

    q_rep = dotD(bd, q)
    scores = dotD(q_rep * k, seg) * (1.0 / (DH ** 0.5))
    scores = scores + mask_add
    e = jnp.exp(scores)
    den = dotD(bdt, e)
    fully_masked = den[:, 0:1] == 0.0
    den = jnp.where(den == 0.0, 1.0, den)
    attn = e * dotD(bd, 1.0 / den)
    av = dotD(attn, seg_t) * v
    outh = dotD(bdt, av)
    out = dotH(outh, wo_ref[...]) + bo_ref[...]
    out = jnp.where(fully_masked, 0.0, out)

    h1 = jnp.maximum(
        dotH(out, fc1a_ref[...]) + dotH(src, fc1b_ref[...]) + fc1b_b_ref[...],
        0.0)
    out_ref[...] = dotH(h1, fc2_ref[...]) + fc2b_ref[...]


def kernel(memory, node_features, edge_features, timestamps, edge_times,
           time_w, time_b, Wq, bq, Wk, bk, Wv, bv, Wo, bo,
           fc1_w, fc1_b, fc2_w, fc2_b, src_nodes, neighbors, edge_idxs):
    f32 = jnp.float32

    combined = pl.pallas_call(
        _combine_body,
        out_shape=jax.ShapeDtypeStruct((N_NODES, NODE_DIM), f32),
        grid=(50,),
        in_specs=[pl.BlockSpec((2000, NODE_DIM), lambda i: (i, 0)),
                  pl.BlockSpec((2000, NODE_DIM), lambda i: (i, 0))],
        out_specs=pl.BlockSpec((2000, NODE_DIM), lambda i: (i, 0)),
    )(memory, node_features)

    flat_nbr = neighbors.reshape(-1).astype(jnp.int32)
    flat_eix = edge_idxs.reshape(-1).astype(jnp.int32)
    node_idx = jnp.concatenate([
        flat_nbr, src_nodes.astype(jnp.int32),
        jnp.zeros((16 * NODE_N1 + NODE_N0 - (B * NBR + B - 16 * NODE_N0),),
                  jnp.int32)])
    ew_idx = jnp.concatenate([
        flat_eix // EPR,
        jnp.zeros((16 * EW_N1 + EW_N0 - (B * NBR - 16 * EW_N0),), jnp.int32)])
    efw = edge_features.reshape(N_EDGES // EPR, NODE_DIM)

    mesh = plsc.VectorSubcoreMesh(core_axis_name="c", subcore_axis_name="s")
    node_rows = pl.kernel(
        _sc_node_body,
        out_type=jax.ShapeDtypeStruct((NODE_TOT, NODE_DIM), f32),
        mesh=mesh,
        scratch_types=[
            pltpu.VMEM((NODE_N0,), jnp.int32),
            pltpu.VMEM((SC_CH, NODE_DIM), f32),
            pltpu.VMEM((SC_CH, NODE_DIM), f32),
            pltpu.SemaphoreType.DMA,
            pltpu.SemaphoreType.DMA,
        ],
    )(combined, node_idx)

    ew_rows = pl.kernel(
        _sc_edge_body,
        out_type=jax.ShapeDtypeStruct((EW_TOT, NODE_DIM), f32),
        mesh=mesh,
        scratch_types=[
            pltpu.VMEM((EW_N0,), jnp.int32),
            pltpu.VMEM((SC_CH, NODE_DIM), f32),
            pltpu.VMEM((SC_CH, NODE_DIM), f32),
            pltpu.SemaphoreType.DMA,
            pltpu.SemaphoreType.DMA,
        ],
    )(efw, ew_idx)

    ts_col = timestamps.reshape(B, 1).astype(f32)
    et2d = edge_times.astype(f32)
    mask2d = jnp.where(neighbors == 0, -1e9, 0.0).astype(f32)
    g2d = (flat_eix.reshape(B, NBR) % EPR).astype(f32)

    tw = time_w.reshape(1, TIME_DIM)
    tb = time_b.reshape(1, TIME_DIM)
    wq1 = Wq[:, :NODE_DIM].T
    wq2 = Wq[:, NODE_DIM:].T
    wk_t = Wk.T
    wv_t = Wv.T
    wo_t = Wo.T
    fc1a = fc1_w[:, :QDIM].T
    fc1b = fc1_w[:, QDIM:].T
    fc2t = fc2_w.T

    def full(a):
        a2 = a.reshape((1, -1)) if a.ndim == 1 else a
        return a2, pl.BlockSpec(a2.shape, lambda i: tuple(0 for _ in a2.shape))

    const_args = [tw, tb, wq1, wq2, bq, wk_t, bk, wv_t, bv,
                  wo_t, bo, fc1a, fc1b, fc1_b, fc2t, fc2_b]
    const_vals, const_specs = zip(*[full(a) for a in const_args])

    out = pl.pallas_call(
        _attn_body,
        out_shape=jax.ShapeDtypeStruct((B, NODE_DIM), f32),
        grid=(NBLK,),
        in_specs=[
            pl.BlockSpec((NRB, NODE_DIM), lambda i: (i, 0)),
            pl.BlockSpec((RB, NODE_DIM), lambda i: (B * NBR // RB + i, 0)),
            pl.BlockSpec((NRB, NODE_DIM), lambda i: (i, 0)),
            pl.BlockSpec((RB, NBR), lambda i: (i, 0)),
            pl.BlockSpec((RB, 1), lambda i: (i, 0)),
            pl.BlockSpec((RB, NBR), lambda i: (i, 0)),
            pl.BlockSpec((RB, NBR), lambda i: (i, 0)),
        ] + list(const_specs),
        out_specs=pl.BlockSpec((RB, NODE_DIM), lambda i: (i, 0)),
    )(node_rows, node_rows, ew_rows, et2d, ts_col, mask2d, g2d,
      *const_vals)
    return out


# speedup vs baseline: 1.1665x; 1.0057x over previous
import functools

import jax
import jax.numpy as jnp
from jax import lax
from jax.experimental import pallas as pl
from jax.experimental.pallas import tpu as pltpu
from jax.experimental.pallas import tpu_sc as plsc

N_NODES = 100000
N_EDGES = 1600000
B = 10000
NBR = 20
NODE_DIM = 128
EDGE_DIM = 16
TIME_DIM = 16
QDIM = NODE_DIM + TIME_DIM
N_HEADS = 2
DH = QDIM // N_HEADS
EPR = NODE_DIM // EDGE_DIM

SC_CH = 320
NODE_TOT = 215040
NODE_PW = NODE_TOT // 16
EW_TOT = 204800
EW_PW = EW_TOT // 16

RB = 80
NRB = RB * NBR
NBLK = B // RB


def _combine_body(m_ref, f_ref, o_ref):
    o_ref[...] = m_ref[...] + f_ref[...]


def _make_sc_body(n_pw):
    nch = n_pw // SC_CH

    def body(table_hbm, idx_hbm, out, idx_v, buf0, buf1, sem0, sem1):
        s = lax.axis_index("s")
        base = s * n_pw
        pltpu.sync_copy(idx_hbm.at[pl.ds(base, n_pw)], idx_v)
        bufs, sems = (buf0, buf1), (sem0, sem1)
        for p in range(2):
            pltpu.async_copy(
                table_hbm.at[idx_v.at[pl.ds(p * SC_CH, SC_CH)]],
                bufs[p], sems[p])

        @pl.loop(0, nch, step=2)
        def _(g):
            for p in range(2):
                cch = g + p
                buf, sem = bufs[p], sems[p]
                pltpu.make_async_copy(
                    table_hbm.at[pl.ds(0, SC_CH)], buf, sem).wait()
                pltpu.sync_copy(
                    buf, out.at[pl.ds(base + cch * SC_CH, SC_CH)])

                @pl.when(cch + 2 < nch)
                def _issue():
                    off = (cch + 2) * SC_CH
                    pltpu.async_copy(
                        table_hbm.at[idx_v.at[pl.ds(off, SC_CH)]],
                        buf, sem)

    return body


_sc_node_body = _make_sc_body(NODE_PW)
_sc_edge_body = _make_sc_body(EW_PW)


def _attn_body(nbr_ref, src_ref, efw_ref, et_ref, ts_ref, mk_ref, g_ref,
               tw_ref, tb_ref,
               wq1_ref, wq2_ref, bq_ref,
               wk_ref, bk_ref, wv_ref, bv_ref,
               wo_ref, bo_ref,
               fc1a_ref, fc1b_ref, fc1b_b_ref, fc2_ref, fc2b_ref,
               out_ref):
    f32 = jnp.float32
    i32 = jnp.int32
    rows = lax.broadcasted_iota(i32, (NRB, RB), 0) // NBR
    cols = lax.broadcasted_iota(i32, (NRB, RB), 1)
    bd = (rows == cols).astype(f32)
    rows_t = lax.broadcasted_iota(i32, (RB, NRB), 1) // NBR
    cols_t = lax.broadcasted_iota(i32, (RB, NRB), 0)
    bdt = (rows_t == cols_t).astype(f32)
    dsel = lax.broadcasted_iota(i32, (QDIM, N_HEADS), 0) // DH
    hsel = lax.broadcasted_iota(i32, (QDIM, N_HEADS), 1)
    seg = (dsel == hsel).astype(f32)
    dsel2 = lax.broadcasted_iota(i32, (N_HEADS, QDIM), 1) // DH
    hsel2 = lax.broadcasted_iota(i32, (N_HEADS, QDIM), 0)
    seg_t = (dsel2 == hsel2).astype(f32)

    dotH = functools.partial(jnp.dot, preferred_element_type=f32,
                             precision=lax.Precision.HIGHEST)
    dotD = functools.partial(jnp.dot, preferred_element_type=f32)

    lanesel = (lax.broadcasted_iota(i32, (NRB, NBR), 1)
               == lax.broadcasted_iota(i32, (NRB, NBR), 0) % NBR)

    def expand_col(x):
        e = jnp.where(lanesel, dotD(bd, x), 0.0)
        return jnp.sum(e, axis=1, keepdims=True)

    def split_hi(x):
        return x.astype(jnp.bfloat16).astype(f32)

    nbr = nbr_ref[...]
    src = src_ref[...]
    efw = efw_ref[...]
    et2 = et_ref[...]
    et_hi = split_hi(et2)
    et = expand_col(et_hi) + expand_col(et2 - et_hi)
    mask_add = expand_col(mk_ref[...])
    gcol = expand_col(g_ref[...])
    ts2 = ts_ref[...]
    ts_hi = split_hi(ts2)
    ts_rep = dotD(bd, ts_hi) + dotD(bd, ts2 - ts_hi)

    lane = lax.broadcasted_iota(i32, (NRB, NODE_DIM), 1)
    ef_sel = jnp.where((lane // EDGE_DIM).astype(f32) == gcol, efw, 0.0)
    psel = (lax.broadcasted_iota(i32, (NODE_DIM, EDGE_DIM), 0) % EDGE_DIM
            == lax.broadcasted_iota(i32, (NODE_DIM, EDGE_DIM), 1))
    ef = dotD(ef_sel, psel.astype(f32))

    deltas = ts_rep - et
    et_enc = jnp.cos(deltas * tw_ref[...] + tb_ref[...])
    st_row = jnp.cos(tb_ref[...])

    q = dotH(src, wq1_ref[...]) + dotH(st_row, wq2_ref[...]) + bq_ref[...]
    keyk = jnp.concatenate([nbr, et_enc, ef], axis=1)
    k = dotH(keyk, wk_ref[...]) + bk_ref[...]
    v = dotH(keyk, wv_ref[...]) + bv_ref[...]

    q_rep = dotD(bd, q)
    scores = dotD(q_rep * k, seg) * (1.0 / (DH ** 0.5))
    scores = scores + mask_add
    e = jnp.exp(scores)
    den = dotD(bdt, e)
    fully_masked = den[:, 0:1] == 0.0
    den = jnp.where(den == 0.0, 1.0, den)
    attn = e * dotD(bd, 1.0 / den)
    av = dotD(attn, seg_t) * v
    outh = dotD(bdt, av)
    out = dotH(outh, wo_ref[...]) + bo_ref[...]
    out = jnp.where(fully_masked, 0.0, out)

    h1 = jnp.maximum(
        dotH(out, fc1a_ref[...]) + dotH(src, fc1b_ref[...]) + fc1b_b_ref[...],
        0.0)
    out_ref[...] = dotH(h1, fc2_ref[...]) + fc2b_ref[...]


def kernel(memory, node_features, edge_features, timestamps, edge_times,
           time_w, time_b, Wq, bq, Wk, bk, Wv, bv, Wo, bo,
           fc1_w, fc1_b, fc2_w, fc2_b, src_nodes, neighbors, edge_idxs):
    f32 = jnp.float32

    combined = pl.pallas_call(
        _combine_body,
        out_shape=jax.ShapeDtypeStruct((N_NODES, NODE_DIM), f32),
        grid=(50,),
        in_specs=[pl.BlockSpec((2000, NODE_DIM), lambda i: (i, 0)),
                  pl.BlockSpec((2000, NODE_DIM), lambda i: (i, 0))],
        out_specs=pl.BlockSpec((2000, NODE_DIM), lambda i: (i, 0)),
    )(memory, node_features)

    flat_nbr = neighbors.reshape(-1).astype(jnp.int32)
    flat_eix = edge_idxs.reshape(-1).astype(jnp.int32)
    node_idx = jnp.concatenate([
        flat_nbr, src_nodes.astype(jnp.int32),
        jnp.zeros((NODE_TOT - B * NBR - B,), jnp.int32)])
    ew_idx = jnp.concatenate([
        flat_eix // EPR, jnp.zeros((EW_TOT - B * NBR,), jnp.int32)])
    efw = edge_features.reshape(N_EDGES // EPR, NODE_DIM)

    mesh = plsc.VectorSubcoreMesh(core_axis_name="c", subcore_axis_name="s",
                                  num_cores=1)
    node_rows = pl.kernel(
        _sc_node_body,
        out_type=jax.ShapeDtypeStruct((NODE_TOT, NODE_DIM), f32),
        mesh=mesh,
        scratch_types=[
            pltpu.VMEM((NODE_PW,), jnp.int32),
            pltpu.VMEM((SC_CH, NODE_DIM), f32),
            pltpu.VMEM((SC_CH, NODE_DIM), f32),
            pltpu.SemaphoreType.DMA,
            pltpu.SemaphoreType.DMA,
        ],
    )(combined, node_idx)

    ew_rows = pl.kernel(
        _sc_edge_body,
        out_type=jax.ShapeDtypeStruct((EW_TOT, NODE_DIM), f32),
        mesh=mesh,
        scratch_types=[
            pltpu.VMEM((EW_PW,), jnp.int32),
            pltpu.VMEM((SC_CH, NODE_DIM), f32),
            pltpu.VMEM((SC_CH, NODE_DIM), f32),
            pltpu.SemaphoreType.DMA,
            pltpu.SemaphoreType.DMA,
        ],
    )(efw, ew_idx)

    ts_col = timestamps.reshape(B, 1).astype(f32)
    et2d = edge_times.astype(f32)
    mask2d = jnp.where(neighbors == 0, -1e9, 0.0).astype(f32)
    g2d = (flat_eix.reshape(B, NBR) % EPR).astype(f32)

    tw = time_w.reshape(1, TIME_DIM)
    tb = time_b.reshape(1, TIME_DIM)
    wq1 = Wq[:, :NODE_DIM].T
    wq2 = Wq[:, NODE_DIM:].T
    wk_t = Wk.T
    wv_t = Wv.T
    wo_t = Wo.T
    fc1a = fc1_w[:, :QDIM].T
    fc1b = fc1_w[:, QDIM:].T
    fc2t = fc2_w.T

    def full(a):
        a2 = a.reshape((1, -1)) if a.ndim == 1 else a
        return a2, pl.BlockSpec(a2.shape, lambda i: tuple(0 for _ in a2.shape))

    const_args = [tw, tb, wq1, wq2, bq, wk_t, bk, wv_t, bv,
                  wo_t, bo, fc1a, fc1b, fc1_b, fc2t, fc2_b]
    const_vals, const_specs = zip(*[full(a) for a in const_args])

    out = pl.pallas_call(
        _attn_body,
        out_shape=jax.ShapeDtypeStruct((B, NODE_DIM), f32),
        grid=(NBLK,),
        in_specs=[
            pl.BlockSpec((NRB, NODE_DIM), lambda i: (i, 0)),
            pl.BlockSpec((RB, NODE_DIM), lambda i: (B * NBR // RB + i, 0)),
            pl.BlockSpec((NRB, NODE_DIM), lambda i: (i, 0)),
            pl.BlockSpec((RB, NBR), lambda i: (i, 0)),
            pl.BlockSpec((RB, 1), lambda i: (i, 0)),
            pl.BlockSpec((RB, NBR), lambda i: (i, 0)),
            pl.BlockSpec((RB, NBR), lambda i: (i, 0)),
        ] + list(const_specs),
        out_specs=pl.BlockSpec((RB, NODE_DIM), lambda i: (i, 0)),
    )(node_rows, node_rows, ew_rows, et2d, ts_col, mask2d, g2d,
      *const_vals)
    return out


# speedup vs baseline: 1.5118x; 1.2960x over previous
import functools

import jax
import jax.numpy as jnp
from jax import lax
from jax.experimental import pallas as pl
from jax.experimental.pallas import tpu as pltpu
from jax.experimental.pallas import tpu_sc as plsc

N_NODES = 100000
N_EDGES = 1600000
B = 10000
NBR = 20
NODE_DIM = 128
EDGE_DIM = 16
TIME_DIM = 16
QDIM = NODE_DIM + TIME_DIM
N_HEADS = 2
DH = QDIM // N_HEADS
EPR = NODE_DIM // EDGE_DIM

SC_CH = 320
NODE_TOT = 215040
NODE_PW = NODE_TOT // 16
EW_TOT = 204800
EW_PW = EW_TOT // 16
EW_CH = 1600

RB = 80
NRB = RB * NBR
NBLK = B // RB


def _combine_body(m_ref, f_ref, o_ref):
    o_ref[...] = m_ref[...] + f_ref[...]


def _make_sc_body(n_pw, ch):
    nch = n_pw // ch

    def body(table_hbm, idx_hbm, out, idx_v, buf0, buf1, sem0, sem1):
        s = lax.axis_index("s")
        base = s * n_pw
        pltpu.sync_copy(idx_hbm.at[pl.ds(base, n_pw)], idx_v)
        bufs, sems = (buf0, buf1), (sem0, sem1)
        for p in range(2):
            pltpu.async_copy(
                table_hbm.at[idx_v.at[pl.ds(p * ch, ch)]],
                bufs[p], sems[p])

        @pl.loop(0, nch, step=2)
        def _(g):
            for p in range(2):
                cch = g + p
                buf, sem = bufs[p], sems[p]
                pltpu.make_async_copy(
                    table_hbm.at[pl.ds(0, ch)], buf, sem).wait()
                pltpu.sync_copy(
                    buf, out.at[pl.ds(base + cch * ch, ch)])

                @pl.when(cch + 2 < nch)
                def _issue():
                    off = (cch + 2) * ch
                    pltpu.async_copy(
                        table_hbm.at[idx_v.at[pl.ds(off, ch)]],
                        buf, sem)

    return body


_sc_node_body = _make_sc_body(NODE_PW, SC_CH)
_sc_edge_body = _make_sc_body(EW_PW, EW_CH)


def _attn_body(nbr_ref, src_ref, ef_ref, et_ref, ts_ref, mk_ref,
               tw_ref, tb_ref,
               wq1_ref, wq2_ref, bq_ref,
               wkh_ref, wkl_ref, bk_ref, wvh_ref, wvl_ref, bv_ref,
               wo_ref, bo_ref,
               fc1a_ref, fc1b_ref, fc1b_b_ref, fc2_ref, fc2b_ref,
               out_ref):
    f32 = jnp.float32
    i32 = jnp.int32
    rows = lax.broadcasted_iota(i32, (NRB, RB), 0) // NBR
    cols = lax.broadcasted_iota(i32, (NRB, RB), 1)
    bd = (rows == cols).astype(f32)
    rows_t = lax.broadcasted_iota(i32, (RB, NRB), 1) // NBR
    cols_t = lax.broadcasted_iota(i32, (RB, NRB), 0)
    bdt = (rows_t == cols_t).astype(f32)
    dsel = lax.broadcasted_iota(i32, (QDIM, N_HEADS), 0) // DH
    hsel = lax.broadcasted_iota(i32, (QDIM, N_HEADS), 1)
    seg = (dsel == hsel).astype(f32)
    dsel2 = lax.broadcasted_iota(i32, (N_HEADS, QDIM), 1) // DH
    hsel2 = lax.broadcasted_iota(i32, (N_HEADS, QDIM), 0)
    seg_t = (dsel2 == hsel2).astype(f32)

    dotH = functools.partial(jnp.dot, preferred_element_type=f32,
                             precision=lax.Precision.HIGHEST)
    dotD = functools.partial(jnp.dot, preferred_element_type=f32)

    lanesel = (lax.broadcasted_iota(i32, (NRB, NBR), 1)
               == lax.broadcasted_iota(i32, (NRB, NBR), 0) % NBR)

    def expand_col(x):
        e = jnp.where(lanesel, dotD(bd, x), 0.0)
        return jnp.sum(e, axis=1, keepdims=True)

    def split_hi(x):
        return x.astype(jnp.bfloat16).astype(f32)

    def fast_cos(x):
        n = jnp.round(x * 0.15915494309189535)
        y = (x - n * 6.28125) - n * 1.9353071795864769e-3
        t = y * y
        p = 1.736911670047192e-09
        p = p * t - 2.7113368709918984e-07
        p = p * t + 2.4773423737313102e-05
        p = p * t - 1.388797038821366e-03
        p = p * t + 4.166652435845298e-02
        p = p * t - 4.999999177167546e-01
        return p * t + 9.999999922847383e-01

    nbr = nbr_ref[...]
    src = src_ref[...]
    ef = ef_ref[...]
    et2 = et_ref[...]
    et_hi = split_hi(et2)
    et = expand_col(et_hi) + expand_col(et2 - et_hi)
    mask_add = expand_col(mk_ref[...])
    ts2 = ts_ref[...]
    ts_hi = split_hi(ts2)
    ts_rep = dotD(bd, ts_hi) + dotD(bd, ts2 - ts_hi)

    deltas = ts_rep - et
    et_enc = fast_cos(deltas * tw_ref[...] + tb_ref[...])
    st_row = fast_cos(tb_ref[...])

    q = dotH(src, wq1_ref[...]) + dotH(st_row, wq2_ref[...]) + bq_ref[...]
    keyk = jnp.concatenate([nbr, et_enc, ef], axis=1)
    keyk_hi = split_hi(keyk)
    keyk_lo = keyk - keyk_hi
    k = (dotD(keyk_hi, wkh_ref[...]) + dotD(keyk_hi, wkl_ref[...])
         + dotD(keyk_lo, wkh_ref[...]) + bk_ref[...])
    v = (dotD(keyk_hi, wvh_ref[...]) + dotD(keyk_hi, wvl_ref[...])
         + dotD(keyk_lo, wvh_ref[...]) + bv_ref[...])

    q_rep = dotD(bd, q)
    scores = dotD(q_rep * k, seg) * (1.0 / (DH ** 0.5))
    scores = scores + mask_add
    e = jnp.exp(scores)
    den = dotD(bdt, e)
    fully_masked = den[:, 0:1] == 0.0
    den = jnp.where(den == 0.0, 1.0, den)
    attn = e * dotD(bd, 1.0 / den)
    av = dotD(attn, seg_t) * v
    outh = dotD(bdt, av)
    out = dotH(outh, wo_ref[...]) + bo_ref[...]
    out = jnp.where(fully_masked, 0.0, out)

    h1 = jnp.maximum(
        dotH(out, fc1a_ref[...]) + dotH(src, fc1b_ref[...]) + fc1b_b_ref[...],
        0.0)
    out_ref[...] = dotH(h1, fc2_ref[...]) + fc2b_ref[...]


def kernel(memory, node_features, edge_features, timestamps, edge_times,
           time_w, time_b, Wq, bq, Wk, bk, Wv, bv, Wo, bo,
           fc1_w, fc1_b, fc2_w, fc2_b, src_nodes, neighbors, edge_idxs):
    f32 = jnp.float32

    combined = pl.pallas_call(
        _combine_body,
        out_shape=jax.ShapeDtypeStruct((N_NODES, NODE_DIM), f32),
        grid=(50,),
        in_specs=[pl.BlockSpec((2000, NODE_DIM), lambda i: (i, 0)),
                  pl.BlockSpec((2000, NODE_DIM), lambda i: (i, 0))],
        out_specs=pl.BlockSpec((2000, NODE_DIM), lambda i: (i, 0)),
    )(memory, node_features)

    flat_nbr = neighbors.reshape(-1).astype(jnp.int32)
    flat_eix = edge_idxs.reshape(-1).astype(jnp.int32)
    node_idx = jnp.concatenate([
        flat_nbr, src_nodes.astype(jnp.int32),
        jnp.zeros((NODE_TOT - B * NBR - B,), jnp.int32)])
    ew_idx = jnp.concatenate([
        flat_eix, jnp.zeros((EW_TOT - B * NBR,), jnp.int32)])

    mesh = plsc.VectorSubcoreMesh(core_axis_name="c", subcore_axis_name="s",
                                  num_cores=1)
    node_rows = pl.kernel(
        _sc_node_body,
        out_type=jax.ShapeDtypeStruct((NODE_TOT, NODE_DIM), f32),
        mesh=mesh,
        scratch_types=[
            pltpu.VMEM((NODE_PW,), jnp.int32),
            pltpu.VMEM((SC_CH, NODE_DIM), f32),
            pltpu.VMEM((SC_CH, NODE_DIM), f32),
            pltpu.SemaphoreType.DMA,
            pltpu.SemaphoreType.DMA,
        ],
    )(combined, node_idx)

    ew_rows = pl.kernel(
        _sc_edge_body,
        out_type=jax.ShapeDtypeStruct((EW_TOT, EDGE_DIM), f32),
        mesh=mesh,
        compiler_params=pltpu.CompilerParams(use_tc_tiling_on_sc=False),
        scratch_types=[
            pltpu.VMEM((EW_PW,), jnp.int32),
            pltpu.VMEM((EW_CH, EDGE_DIM), f32),
            pltpu.VMEM((EW_CH, EDGE_DIM), f32),
            pltpu.SemaphoreType.DMA,
            pltpu.SemaphoreType.DMA,
        ],
    )(edge_features, ew_idx)

    ts_col = timestamps.reshape(B, 1).astype(f32)
    et2d = edge_times.astype(f32)
    mask2d = jnp.where(neighbors == 0, -1e9, 0.0).astype(f32)

    tw = time_w.reshape(1, TIME_DIM)
    tb = time_b.reshape(1, TIME_DIM)
    wq1 = Wq[:, :NODE_DIM].T
    wq2 = Wq[:, NODE_DIM:].T
    wk_t = Wk.T
    wv_t = Wv.T
    wk_hi = wk_t.astype(jnp.bfloat16).astype(f32)
    wk_lo = wk_t - wk_hi
    wv_hi = wv_t.astype(jnp.bfloat16).astype(f32)
    wv_lo = wv_t - wv_hi
    wo_t = Wo.T
    fc1a = fc1_w[:, :QDIM].T
    fc1b = fc1_w[:, QDIM:].T
    fc2t = fc2_w.T

    def full(a):
        a2 = a.reshape((1, -1)) if a.ndim == 1 else a
        return a2, pl.BlockSpec(a2.shape, lambda i: tuple(0 for _ in a2.shape))

    const_args = [tw, tb, wq1, wq2, bq, wk_hi, wk_lo, bk, wv_hi, wv_lo, bv,
                  wo_t, bo, fc1a, fc1b, fc1_b, fc2t, fc2_b]
    const_vals, const_specs = zip(*[full(a) for a in const_args])

    out = pl.pallas_call(
        _attn_body,
        out_shape=jax.ShapeDtypeStruct((B, NODE_DIM), f32),
        grid=(NBLK,),
        in_specs=[
            pl.BlockSpec((NRB, NODE_DIM), lambda i: (i, 0)),
            pl.BlockSpec((RB, NODE_DIM), lambda i: (B * NBR // RB + i, 0)),
            pl.BlockSpec((NRB, EDGE_DIM), lambda i: (i, 0)),
            pl.BlockSpec((RB, NBR), lambda i: (i, 0)),
            pl.BlockSpec((RB, 1), lambda i: (i, 0)),
            pl.BlockSpec((RB, NBR), lambda i: (i, 0)),
        ] + list(const_specs),
        out_specs=pl.BlockSpec((RB, NODE_DIM), lambda i: (i, 0)),
    )(node_rows, node_rows, ew_rows, et2d, ts_col, mask2d,
      *const_vals)
    return out
